# baseline (speedup 1.0000x reference)
import functools

import jax
import jax.numpy as jnp
from jax import lax
from jax.experimental import pallas as pl
from jax.experimental.pallas import tpu as pltpu
from jax.experimental.pallas import tpu_sc as plsc

_NC = 2
_NS = 16
_NW = _NC * _NS

_BATCH = 16384
_DIM = 32
_PACK = 8
_PROWS = 1000000 // _PACK
_PW = _DIM * _PACK
_B_PER_W = _BATCH // _NW
_CHUNK = 128
_NCHUNK = _B_PER_W // _CHUNK


def _extract_chunk(rows4, sub_v, outT, lane):
    def group(g, _):
        ent = lane + g * 16
        sub = sub_v[pl.ds(g * 16, 16)]
        colbase = sub * _DIM

        def comp(c, _):
            vals = plsc.load_gather(rows4, [ent, colbase + c])
            outT[c, pl.ds(g * 16, 16)] = vals
            return _

        return lax.fori_loop(0, _DIM, comp, _, unroll=4)

    lax.fori_loop(0, _CHUNK // 16, group, 0, unroll=False)


def _gather_body(uj_hbm, us_hbm, ij_hbm, is_hbm, up_hbm, ip_hbm,
                 ueT_hbm, ieT_hbm,
                 j_v, s_v, rows4, outT, sem):
    wid = lax.axis_index("s") * _NC + lax.axis_index("c")
    lane = lax.iota(jnp.int32, 16)

    def table(jh, sh, ph, oh):
        def chunk(k, _):
            base = wid * _B_PER_W + k * _CHUNK
            pltpu.sync_copy(jh.at[pl.ds(base, _CHUNK)], j_v)
            pltpu.sync_copy(sh.at[pl.ds(base, _CHUNK)], s_v)
            pltpu.async_copy(ph.at[j_v], rows4, sem).wait()
            _extract_chunk(rows4, s_v, outT, lane)
            pltpu.sync_copy(outT, oh.at[:, pl.ds(base, _CHUNK)])
            return _

        lax.fori_loop(0, _NCHUNK, chunk, 0, unroll=False)

    table(uj_hbm, us_hbm, up_hbm, ueT_hbm)
    table(ij_hbm, is_hbm, ip_hbm, ieT_hbm)


_gather = pl.kernel(
    _gather_body,
    out_type=(
        jax.ShapeDtypeStruct((_DIM, _BATCH), jnp.float32),
        jax.ShapeDtypeStruct((_DIM, _BATCH), jnp.float32),
    ),
    mesh=plsc.VectorSubcoreMesh(
        core_axis_name="c", subcore_axis_name="s",
        num_cores=_NC, num_subcores=_NS),
    scratch_types=(
        pltpu.VMEM((_CHUNK,), jnp.int32),
        pltpu.VMEM((_CHUNK,), jnp.int32),
        pltpu.VMEM((_CHUNK, _PW), jnp.float32),
        pltpu.VMEM((_DIM, _CHUNK), jnp.float32),
        pltpu.SemaphoreType.DMA,
    ),
    compiler_params=pltpu.CompilerParams(needs_layout_passes=False),
)

_BB = 1024


def _mlp_body(ueT_ref, ieT_ref, w1u_ref, w1i_ref, b1_ref, w2_ref, b2_ref,
              w3t_ref, b3_ref, out_ref):
    dn = (((0,), (0,)), ((), ()))
    h = lax.dot_general(ueT_ref[...], w1u_ref[...], dn,
                        preferred_element_type=jnp.float32)
    h = h + lax.dot_general(ieT_ref[...], w1i_ref[...], dn,
                            preferred_element_type=jnp.float32)
    h = jnp.maximum(h + b1_ref[...], 0.0)
    h = jnp.maximum(
        jnp.dot(h, w2_ref[...], preferred_element_type=jnp.float32)
        + b2_ref[...], 0.0)
    out_ref[...] = jnp.sum(h * w3t_ref[...], axis=1) + b3_ref[0, 0]


def _mlp(ueT, ieT, w1u, w1i, b1, w2, b2, w3t, b3):
    grid = _BATCH // _BB
    full = lambda s: pl.BlockSpec(s, lambda i: (0,) * len(s))
    return pl.pallas_call(
        _mlp_body,
        grid=(grid,),
        in_specs=[
            pl.BlockSpec((_DIM, _BB), lambda i: (0, i)),
            pl.BlockSpec((_DIM, _BB), lambda i: (0, i)),
            full((_DIM, 128)),
            full((_DIM, 128)),
            full((1, 128)),
            full((128, 64)),
            full((1, 64)),
            full((1, 64)),
            full((1, 1)),
        ],
        out_specs=pl.BlockSpec((_BB,), lambda i: (i,)),
        out_shape=jax.ShapeDtypeStruct((_BATCH,), jnp.float32),
        compiler_params=pltpu.CompilerParams(
            dimension_semantics=("arbitrary",)),
    )(ueT, ieT, w1u, w1i, b1, w2, b2, w3t, b3)


@jax.jit
def kernel(user_idx, item_idx, user_table, item_table, W1, b1, W2, b2, W3, b3):
    ui = user_idx.astype(jnp.int32)
    ii = item_idx.astype(jnp.int32)
    up = user_table.reshape(_PROWS, _PW)
    ip = item_table.reshape(_PROWS, _PW)
    ueT, ieT = _gather(ui // _PACK, ui % _PACK, ii // _PACK, ii % _PACK,
                       up, ip)
    return _mlp(ueT, ieT, W1[:_DIM], W1[_DIM:], b1.reshape(1, 128),
                W2, b2.reshape(1, 64), W3.reshape(1, 64), b3.reshape(1, 1))
